# baseline (speedup 1.0000x reference)
import jax
import jax.numpy as jnp
from jax.experimental import pallas as pl
from jax.experimental.pallas import tpu as pltpu

_MAX_LENGTH = 128
_N_CHUNKS = 8


def _resample_body(x_ref, o_ref, *sems):
    L = o_ref.shape[0]
    rows = L - 1
    per = rows // _N_CHUNKS
    copies = []
    start = 0
    for c in range(_N_CHUNKS):
        n = per + (1 if c < rows % _N_CHUNKS else 0)
        copies.append(
            pltpu.make_async_copy(
                x_ref.at[pl.ds(start, n), 0, :, :],
                o_ref.at[pl.ds(start, n), :, :],
                sems[c],
            )
        )
        start += n
    copies.append(
        pltpu.make_async_copy(
            x_ref.at[L - 1, 1, :, :], o_ref.at[L - 1, :, :], sems[_N_CHUNKS]
        )
    )
    for c in copies:
        c.start()
    for c in copies:
        c.wait()


def kernel(x):
    T, C, H, W = x.shape
    L = _MAX_LENGTH
    F = C * H * W
    x4 = x.reshape(L, T // L, F // 128, 128)

    out = pl.pallas_call(
        _resample_body,
        in_specs=[pl.BlockSpec(memory_space=pl.ANY)],
        out_specs=pl.BlockSpec(memory_space=pl.ANY),
        out_shape=jax.ShapeDtypeStruct((L, F // 128, 128), x.dtype),
        scratch_shapes=[pltpu.SemaphoreType.DMA] * (_N_CHUNKS + 1),
    )(x4)
    return out.reshape(L, C, H, W)


# speedup vs baseline: 5.6639x; 5.6639x over previous
import jax
import jax.numpy as jnp
from jax.experimental import pallas as pl
from jax.experimental.pallas import tpu as pltpu

_MAX_LENGTH = 128
_BLOCK = 8


def _even_body(x_ref, o_ref):
    o_ref[...] = x_ref[:, 0]


def _fixup_body(x_ref, main_ref, out_ref, sem):
    L = out_ref.shape[0]
    cp = pltpu.make_async_copy(x_ref.at[L - 1, 1], out_ref.at[L - 1], sem)
    cp.start()
    cp.wait()


def kernel(x):
    T, C, H, W = x.shape
    L = _MAX_LENGTH
    F = C * H * W
    R = F // 128
    x4 = x.reshape(L, T // L, R, 128)

    main = pl.pallas_call(
        _even_body,
        grid=(L // _BLOCK,),
        in_specs=[pl.BlockSpec((_BLOCK, 1, R, 128), lambda i: (i, 0, 0, 0))],
        out_specs=pl.BlockSpec((_BLOCK, R, 128), lambda i: (i, 0, 0)),
        out_shape=jax.ShapeDtypeStruct((L, R, 128), x.dtype),
    )(x4)

    out = pl.pallas_call(
        _fixup_body,
        in_specs=[
            pl.BlockSpec(memory_space=pl.ANY),
            pl.BlockSpec(memory_space=pl.ANY),
        ],
        out_specs=pl.BlockSpec(memory_space=pl.ANY),
        out_shape=jax.ShapeDtypeStruct((L, R, 128), x.dtype),
        scratch_shapes=[pltpu.SemaphoreType.DMA],
        input_output_aliases={1: 0},
    )(x4, main)
    return out.reshape(L, C, H, W)


# speedup vs baseline: 9.3576x; 1.6522x over previous
import jax
import jax.numpy as jnp
from jax.experimental import pallas as pl
from jax.experimental.pallas import tpu as pltpu

_MAX_LENGTH = 128
_BLOCK = 8


def _even_body(x_ref, o_ref):
    o_ref[...] = x_ref[:, 0]


def _fixup_body(x_ref, main_ref, out_ref, sem):
    L = out_ref.shape[0]
    cp = pltpu.make_async_copy(x_ref.at[L - 1, 1], out_ref.at[L - 1], sem)
    cp.start()
    cp.wait()


def kernel(x):
    T, C, H, W = x.shape
    L = _MAX_LENGTH
    x5 = x.reshape(L, T // L, C, H, W)

    main = pl.pallas_call(
        _even_body,
        grid=(L // _BLOCK,),
        in_specs=[
            pl.BlockSpec((_BLOCK, 1, C, H, W), lambda i: (i, 0, 0, 0, 0))
        ],
        out_specs=pl.BlockSpec((_BLOCK, C, H, W), lambda i: (i, 0, 0, 0)),
        out_shape=jax.ShapeDtypeStruct((L, C, H, W), x.dtype),
    )(x5)

    out = pl.pallas_call(
        _fixup_body,
        in_specs=[
            pl.BlockSpec(memory_space=pl.ANY),
            pl.BlockSpec(memory_space=pl.ANY),
        ],
        out_specs=pl.BlockSpec(memory_space=pl.ANY),
        out_shape=jax.ShapeDtypeStruct((L, C, H, W), x.dtype),
        scratch_shapes=[pltpu.SemaphoreType.DMA],
        input_output_aliases={1: 0},
    )(x5, main)
    return out


# speedup vs baseline: 9.4092x; 1.0055x over previous
import jax
import jax.numpy as jnp
from jax.experimental import pallas as pl
from jax.experimental.pallas import tpu as pltpu

_MAX_LENGTH = 128
_BLOCK = 16


def _even_body(x_ref, o_ref):
    o_ref[...] = x_ref[:, 0]


def _fixup_body(x_ref, main_ref, out_ref, sem):
    L = out_ref.shape[0]
    cp = pltpu.make_async_copy(x_ref.at[L - 1, 1], out_ref.at[L - 1], sem)
    cp.start()
    cp.wait()


def kernel(x):
    T, C, H, W = x.shape
    L = _MAX_LENGTH
    x5 = x.reshape(L, T // L, C, H, W)

    main = pl.pallas_call(
        _even_body,
        grid=(L // _BLOCK,),
        in_specs=[
            pl.BlockSpec((_BLOCK, 1, C, H, W), lambda i: (i, 0, 0, 0, 0))
        ],
        out_specs=pl.BlockSpec((_BLOCK, C, H, W), lambda i: (i, 0, 0, 0)),
        out_shape=jax.ShapeDtypeStruct((L, C, H, W), x.dtype),
    )(x5)

    out = pl.pallas_call(
        _fixup_body,
        in_specs=[
            pl.BlockSpec(memory_space=pl.ANY),
            pl.BlockSpec(memory_space=pl.ANY),
        ],
        out_specs=pl.BlockSpec(memory_space=pl.ANY),
        out_shape=jax.ShapeDtypeStruct((L, C, H, W), x.dtype),
        scratch_shapes=[pltpu.SemaphoreType.DMA],
        input_output_aliases={1: 0},
    )(x5, main)
    return out


# speedup vs baseline: 10.1439x; 1.0781x over previous
import jax
import jax.numpy as jnp
from jax.experimental import pallas as pl
from jax.experimental.pallas import tpu as pltpu

_MAX_LENGTH = 128
_BLOCK = 16


def _body(x_ref, last_ref, o_ref):
    o_ref[...] = x_ref[:, 0]

    @pl.when(pl.program_id(0) == pl.num_programs(0) - 1)
    def _():
        o_ref[o_ref.shape[0] - 1] = last_ref[0, 0]


def kernel(x):
    T, C, H, W = x.shape
    L = _MAX_LENGTH
    x5 = x.reshape(L, T // L, C, H, W)

    out = pl.pallas_call(
        _body,
        grid=(L // _BLOCK,),
        in_specs=[
            pl.BlockSpec((_BLOCK, 1, C, H, W), lambda i: (i, 0, 0, 0, 0)),
            pl.BlockSpec((1, 1, C, H, W), lambda i: (L - 1, 1, 0, 0, 0)),
        ],
        out_specs=pl.BlockSpec((_BLOCK, C, H, W), lambda i: (i, 0, 0, 0)),
        out_shape=jax.ShapeDtypeStruct((L, C, H, W), x.dtype),
    )(x5, x5)
    return out


# speedup vs baseline: 16.8277x; 1.6589x over previous
import jax
import jax.numpy as jnp
from jax.experimental import pallas as pl

_MAX_LENGTH = 128
_BLOCK_M = 1536


def _select_body(a_ref, o_ref):
    t = a_ref.shape[1]
    l = o_ref.shape[1]
    r = jax.lax.broadcasted_iota(jnp.int32, (t, l), 0)
    c = jax.lax.broadcasted_iota(jnp.int32, (t, l), 1)
    g = (r == (c * (t - 1)) // (l - 1)).astype(jnp.float32)
    o_ref[...] = jax.lax.dot_general(
        a_ref[...],
        g,
        (((1,), (0,)), ((), ())),
        precision=jax.lax.Precision.HIGHEST,
        preferred_element_type=jnp.float32,
    )


def kernel(x):
    T, C, H, W = x.shape
    L = _MAX_LENGTH
    M = C * H * W
    xt = jnp.transpose(x, (1, 2, 3, 0)).reshape(M, T)

    out2 = pl.pallas_call(
        _select_body,
        grid=(M // _BLOCK_M,),
        in_specs=[pl.BlockSpec((_BLOCK_M, T), lambda i: (i, 0))],
        out_specs=pl.BlockSpec((_BLOCK_M, L), lambda i: (i, 0)),
        out_shape=jax.ShapeDtypeStruct((M, L), x.dtype),
    )(xt)
    return jnp.transpose(out2.reshape(C, H, W, L), (3, 0, 1, 2))


# speedup vs baseline: 20.3858x; 1.2114x over previous
import jax
import jax.numpy as jnp
from jax.experimental import pallas as pl

_MAX_LENGTH = 128
_BLOCK_M = 3072


def _select_body(a_ref, o_ref):
    t = a_ref.shape[1]
    l = o_ref.shape[1]
    r = jax.lax.broadcasted_iota(jnp.int32, (t, l), 0)
    c = jax.lax.broadcasted_iota(jnp.int32, (t, l), 1)
    g = (r == (c * (t - 1)) // (l - 1)).astype(jnp.float32)
    o_ref[...] = jax.lax.dot_general(
        a_ref[...],
        g,
        (((1,), (0,)), ((), ())),
        precision=jax.lax.Precision.HIGHEST,
        preferred_element_type=jnp.float32,
    )


def kernel(x):
    T, C, H, W = x.shape
    L = _MAX_LENGTH
    M = C * H * W
    xt = jnp.transpose(x, (1, 2, 3, 0)).reshape(M, T)

    out2 = pl.pallas_call(
        _select_body,
        grid=(M // _BLOCK_M,),
        in_specs=[pl.BlockSpec((_BLOCK_M, T), lambda i: (i, 0))],
        out_specs=pl.BlockSpec((_BLOCK_M, L), lambda i: (i, 0)),
        out_shape=jax.ShapeDtypeStruct((M, L), x.dtype),
    )(xt)
    return jnp.transpose(out2.reshape(C, H, W, L), (3, 0, 1, 2))


# speedup vs baseline: 21.5282x; 1.0560x over previous
import jax
import jax.numpy as jnp
from jax.experimental import pallas as pl

_MAX_LENGTH = 128
_BLOCK_M = 6144


def _select_body(a_ref, o_ref):
    t = a_ref.shape[1]
    l = o_ref.shape[1]
    r = jax.lax.broadcasted_iota(jnp.int32, (t, l), 0)
    c = jax.lax.broadcasted_iota(jnp.int32, (t, l), 1)
    g = (r == (c * (t - 1)) // (l - 1)).astype(jnp.float32)
    o_ref[...] = jax.lax.dot_general(
        a_ref[...],
        g,
        (((1,), (0,)), ((), ())),
        precision=jax.lax.Precision.HIGHEST,
        preferred_element_type=jnp.float32,
    )


def kernel(x):
    T, C, H, W = x.shape
    L = _MAX_LENGTH
    M = C * H * W
    xt = jnp.transpose(x, (1, 2, 3, 0)).reshape(M, T)

    out2 = pl.pallas_call(
        _select_body,
        grid=(M // _BLOCK_M,),
        in_specs=[pl.BlockSpec((_BLOCK_M, T), lambda i: (i, 0))],
        out_specs=pl.BlockSpec((_BLOCK_M, L), lambda i: (i, 0)),
        out_shape=jax.ShapeDtypeStruct((M, L), x.dtype),
    )(xt)
    return jnp.transpose(out2.reshape(C, H, W, L), (3, 0, 1, 2))
